# baseline (speedup 1.0000x reference)
import functools

import jax
import jax.numpy as jnp
from jax import lax
from jax.experimental import pallas as pl
from jax.experimental.pallas import tpu as pltpu
from jax.experimental.pallas import tpu_sc as plsc

NC = 2
NS = 16
NW = NC * NS
CH = 128

_PARAMS = pltpu.CompilerParams(use_tc_tiling_on_sc=True, needs_layout_passes=False)
_MESH = dict(core_axis_name="c", subcore_axis_name="s")


def _compact_table(wt, tail_c):
    D, V = wt.shape
    n_full = V // 128
    tail = V - n_full * 128
    Vp = n_full * 128 + (256 if tail else 0)
    per_w = n_full // NW

    mesh = plsc.VectorSubcoreMesh(**_MESH)

    @functools.partial(
        pl.kernel,
        mesh=mesh,
        compiler_params=_PARAMS,
        out_type=jax.ShapeDtypeStruct((Vp // 2, 128), jnp.float32),
        scratch_types=(
            [pltpu.VMEM((2, D, 128), jnp.float32)]
            + [pltpu.VMEM((2, 64, 128), jnp.float32)]
            + [pltpu.SemaphoreType.DMA] * 4
        ),
    )
    def conv(wt_hbm, tail_hbm, out_hbm, tbuf, obuf, tsem0, tsem1, osem0, osem1):
        wid = lax.axis_index("s") * NC + lax.axis_index("c")
        tsem = (tsem0, tsem1)
        osem = (osem0, osem1)

        def blk(k):
            return k * NW + wid

        n_vis = per_w + 1

        def start_read(i_blk, sl):
            pltpu.async_copy(
                wt_hbm.at[:, pl.ds(i_blk * 128, 128)], tbuf.at[sl], tsem[sl]
            )

        def wait_read(i_blk, sl):
            pltpu.make_async_copy(
                wt_hbm.at[:, pl.ds(i_blk * 128, 128)], tbuf.at[sl], tsem[sl]
            ).wait()

        def start_write(i_blk, sl):
            pltpu.async_copy(
                obuf.at[sl], out_hbm.at[pl.ds(i_blk * 64, 64)], osem[sl]
            )

        def wait_write(i_blk, sl):
            pltpu.make_async_copy(
                obuf.at[sl], out_hbm.at[pl.ds(i_blk * 64, 64)], osem[sl]
            ).wait()

        iota = lax.iota(jnp.int32, 16)
        parv = (iota & 1) * 64
        rowvs = [(g * 16 + iota) >> 1 for g in range(8)]

        def transpose(sl):
            def fbody(f, carry):
                colv = parv + f
                for g in range(8):
                    vals = tbuf[sl, f, pl.ds(g * 16, 16)]
                    plsc.store_scatter(obuf.at[sl], [rowvs[g], colv], vals)
                return carry

            lax.fori_loop(0, D, fbody, 0, unroll=False)

        @pl.when(blk(0) < n_full)
        def _():
            start_read(blk(0), 0)

        @pl.when(blk(1) < n_full)
        def _():
            start_read(blk(1), 1)

        def visit(k, sl):
            i_blk = blk(k)

            @pl.when(i_blk < n_full)
            def _():
                wait_read(i_blk, sl)

                @pl.when(k >= 2)
                def _():
                    wait_write(blk(k - 2), sl)

                transpose(sl)
                start_write(i_blk, sl)

                @pl.when(blk(k + 2) < n_full)
                def _():
                    start_read(blk(k + 2), sl)

        def pairvisit(kk, carry):
            for sl in range(2):
                visit(kk * 2 + sl, sl)
            return carry

        lax.fori_loop(0, (n_vis + 2) // 2, pairvisit, 0, unroll=False)

        def pairdrain(kk, carry):
            for sl in range(2):
                k = kk * 2 + sl

                @pl.when((blk(k) < n_full) & (blk(k + 2) >= n_full))
                def _():
                    wait_write(blk(k), sl)

            return carry

        lax.fori_loop(0, (n_vis + 2) // 2, pairdrain, 0, unroll=False)

        if tail:

            @pl.when(wid == 0)
            def _():
                pltpu.sync_copy(tail_hbm, obuf.at[0, pl.ds(0, tail // 2)])
                pltpu.sync_copy(
                    obuf.at[0, pl.ds(0, tail // 2)],
                    out_hbm.at[pl.ds(n_full * 64, tail // 2)],
                )

    return conv(wt, tail_c)


def _gather(table2, idx):
    N = idx.shape[0]
    n_per_w = N // NW
    n_ch = n_per_w // CH
    NBUF = 4
    LA = 2
    WB = 2

    mesh = plsc.VectorSubcoreMesh(**_MESH)

    @functools.partial(
        pl.kernel,
        mesh=mesh,
        compiler_params=_PARAMS,
        out_type=jax.ShapeDtypeStruct((200, 64, 4096), jnp.float32),
        scratch_types=(
            [
                pltpu.VMEM((n_per_w,), jnp.int32),
                pltpu.VMEM((NBUF, CH), jnp.int32),
                pltpu.VMEM((NBUF, CH, 128), jnp.float32),
                pltpu.VMEM((WB, 64, 128), jnp.float32),
            ]
            + [pltpu.SemaphoreType.DMA] * (NBUF + WB)
        ),
    )
    def gat(tab_hbm, idx_hbm, out_hbm, idx_v, pidx, rows, wbuf, *sems):
        gsem = sems[:NBUF]
        osem = sems[NBUF:]
        wid = lax.axis_index("s") * NC + lax.axis_index("c")
        base = wid * n_per_w
        pltpu.sync_copy(idx_hbm.at[pl.ds(base, n_per_w)], idx_v)

        iota = lax.iota(jnp.int32, 16)
        rowvs = [g * 16 + iota for g in range(8)]

        def start_gather(g, b):
            for q in range(8):
                iv = idx_v[pl.ds(g * CH + q * 16, 16)]
                pidx[b, pl.ds(q * 16, 16)] = iv >> 1
            pltpu.async_copy(tab_hbm.at[pidx.at[b]], rows.at[b], gsem[b])

        def wait_gather(g, b):
            pltpu.make_async_copy(
                tab_hbm.at[pidx.at[b]], rows.at[b], gsem[b]
            ).wait()

        def out_ref(g):
            u = base // CH + g
            s = u // 32
            bb = u % 32
            return out_hbm.at[s, :, pl.ds(bb * CH, CH)]

        def start_write(g, ws):
            pltpu.async_copy(wbuf.at[ws], out_ref(g), osem[ws])

        def wait_write(g, ws):
            pltpu.make_async_copy(wbuf.at[ws], out_ref(g), osem[ws]).wait()

        def transpose(g, b, ws):
            def qbody(q, carry):
                iv = idx_v[pl.ds(g * CH + q * 16, 16)]
                parv = (iv & 1) * 64
                rowv = q * 16 + iota
                for f in range(64):
                    vals = plsc.load_gather(rows.at[b], [rowv, parv + f])
                    wbuf[ws, f, pl.ds(q * 16, 16)] = vals
                return carry

            lax.fori_loop(0, 8, qbody, 0, unroll=False)

        def do_visit(g, b, ws, issue, reuse_w):
            wait_gather(g, b)
            if reuse_w:
                wait_write(g - WB, ws)
            transpose(g, b, ws)
            start_write(g, ws)
            if issue:
                start_gather(g + LA, (b + LA) % NBUF)

        for g in range(LA):
            start_gather(g, g % NBUF)

        for g in range(NBUF):
            do_visit(g, g % NBUF, g % WB, g + LA < n_ch, g >= WB)

        def block(blkk, carry):
            for b in range(NBUF):
                g = blkk * NBUF + b
                do_visit(g, b, b % WB, True, True)
            return carry

        lax.fori_loop(1, n_ch // NBUF - 1, block, 0, unroll=False)

        for g in range(n_ch - NBUF, n_ch):
            do_visit(g, g % NBUF, g % WB, g + LA < n_ch, True)

        for g in range(n_ch - WB, n_ch):
            wait_write(g, g % WB)

    return gat(table2, idx)


def kernel(batch_sent_input, embed_weight):
    B, S = batch_sent_input.shape
    ids = jnp.transpose(batch_sent_input).reshape(B * S).astype(jnp.int32)
    wt = jnp.transpose(embed_weight)
    V, D = embed_weight.shape
    tail = V % 128
    tail_c = jnp.reshape(embed_weight[V - tail :, :], (tail // 2, 2 * D))
    table2 = _compact_table(wt, tail_c)
    out_p = _gather(table2, ids)
    return jnp.transpose(out_p, (2, 0, 1))


# speedup vs baseline: 1.9986x; 1.9986x over previous
import functools

import jax
import jax.numpy as jnp
from jax import lax
from jax.experimental import pallas as pl
from jax.experimental.pallas import tpu as pltpu
from jax.experimental.pallas import tpu_sc as plsc

NC = 2
NS = 16
NW = NC * NS
CH = 128
NBUF = 8
LA = 4


def _encode(idx, table):
    N = idx.shape[0]
    D = table.shape[1]
    n_per_w = N // NW
    n_ch = n_per_w // CH

    mesh = plsc.VectorSubcoreMesh(core_axis_name="c", subcore_axis_name="s")

    @functools.partial(
        pl.kernel,
        mesh=mesh,
        compiler_params=pltpu.CompilerParams(use_tc_tiling_on_sc=False),
        out_type=jax.ShapeDtypeStruct((N, D), jnp.float32),
        scratch_types=(
            [
                pltpu.VMEM((n_per_w,), jnp.int32),
                pltpu.VMEM((NBUF, CH, D), jnp.float32),
            ]
            + [pltpu.SemaphoreType.DMA] * (2 * NBUF)
        ),
    )
    def enc(table_hbm, idx_hbm, out_hbm, idx_v, rows_v, *sems):
        gsem = sems[:NBUF]
        osem = sems[NBUF:]
        wid = lax.axis_index("s") * NC + lax.axis_index("c")
        base = wid * n_per_w
        pltpu.sync_copy(idx_hbm.at[pl.ds(base, n_per_w)], idx_v)

        def start_gather(j, b):
            pltpu.async_copy(
                table_hbm.at[idx_v.at[pl.ds(j * CH, CH)]],
                rows_v.at[b],
                gsem[b],
            )

        def wait_gather(j, b):
            pltpu.make_async_copy(
                table_hbm.at[idx_v.at[pl.ds(j * CH, CH)]],
                rows_v.at[b],
                gsem[b],
            ).wait()

        def start_write(j, b):
            pltpu.async_copy(
                rows_v.at[b],
                out_hbm.at[pl.ds(base + j * CH, CH)],
                osem[b],
            )

        def wait_write(j, b):
            pltpu.make_async_copy(
                rows_v.at[b],
                out_hbm.at[pl.ds(base + j * CH, CH)],
                osem[b],
            ).wait()

        def visit(g, b, issue, reuse):
            wait_gather(g, b)
            start_write(g, b)
            if issue:
                jj = g + LA
                bb = (b + LA) % NBUF
                if reuse:
                    wait_write(jj - NBUF, bb)
                start_gather(jj, bb)

        for g in range(LA):
            start_gather(g, g % NBUF)

        for g in range(NBUF):
            visit(g, g % NBUF, g + LA < n_ch, g + LA >= NBUF)

        def block(blk, carry):
            for b in range(NBUF):
                g = blk * NBUF + b
                visit(g, b, True, True)
            return carry

        lax.fori_loop(1, n_ch // NBUF - 1, block, 0)

        for g in range(n_ch - NBUF, n_ch):
            visit(g, g % NBUF, g + LA < n_ch, True)

        for g in range(n_ch - NBUF, n_ch):
            wait_write(g, g % NBUF)

    return enc(table, idx)


def kernel(batch_sent_input, embed_weight):
    B, S = batch_sent_input.shape
    D = embed_weight.shape[1]
    idx = batch_sent_input.reshape(B * S).astype(jnp.int32)
    out = _encode(idx, embed_weight)
    return out.reshape(B, S, D)


# speedup vs baseline: 3.2427x; 1.6224x over previous
import functools

import jax
import jax.numpy as jnp
from jax import lax
from jax.experimental import pallas as pl
from jax.experimental.pallas import tpu as pltpu
from jax.experimental.pallas import tpu_sc as plsc

NC = 2
NS = 16
NW = NC * NS
CH = 128

_PARAMS = pltpu.CompilerParams(use_tc_tiling_on_sc=True, needs_layout_passes=False)
_MESH = dict(core_axis_name="c", subcore_axis_name="s")


def _compact_table(wt, tail_c):
    D, V = wt.shape
    n_full = V // 128
    tail = V - n_full * 128
    Vp = n_full * 128 + (256 if tail else 0)
    per_w = n_full // NW

    mesh = plsc.VectorSubcoreMesh(**_MESH)

    @functools.partial(
        pl.kernel,
        mesh=mesh,
        compiler_params=_PARAMS,
        out_type=jax.ShapeDtypeStruct((Vp // 2, 128), jnp.float32),
        scratch_types=(
            [pltpu.VMEM((2, D, 128), jnp.float32)]
            + [pltpu.VMEM((2, 64, 128), jnp.float32)]
            + [pltpu.SemaphoreType.DMA] * 4
        ),
    )
    def conv(wt_hbm, tail_hbm, out_hbm, tbuf, obuf, tsem0, tsem1, osem0, osem1):
        wid = lax.axis_index("s") * NC + lax.axis_index("c")
        tsem = (tsem0, tsem1)
        osem = (osem0, osem1)

        def blk(k):
            return k * NW + wid

        n_vis = per_w + 1

        def start_read(i_blk, sl):
            pltpu.async_copy(
                wt_hbm.at[:, pl.ds(i_blk * 128, 128)], tbuf.at[sl], tsem[sl]
            )

        def wait_read(i_blk, sl):
            pltpu.make_async_copy(
                wt_hbm.at[:, pl.ds(i_blk * 128, 128)], tbuf.at[sl], tsem[sl]
            ).wait()

        def start_write(i_blk, sl):
            pltpu.async_copy(
                obuf.at[sl], out_hbm.at[pl.ds(i_blk * 64, 64)], osem[sl]
            )

        def wait_write(i_blk, sl):
            pltpu.make_async_copy(
                obuf.at[sl], out_hbm.at[pl.ds(i_blk * 64, 64)], osem[sl]
            ).wait()

        iota = lax.iota(jnp.int32, 16)
        parv = (iota & 1) * 64
        rowvs = [(g * 16 + iota) >> 1 for g in range(8)]
        colrs = [g * 16 + iota for g in range(8)]

        def transpose(sl):
            def fbody(f, carry):
                diag = (f + iota) & (D - 1)
                colw = parv + diag
                vals = [
                    plsc.load_gather(tbuf.at[sl], [diag, colrs[g]])
                    for g in range(8)
                ]
                for g in range(8):
                    plsc.store_scatter(obuf.at[sl], [rowvs[g], colw], vals[g])
                return carry

            lax.fori_loop(0, D, fbody, 0, unroll=False)

        @pl.when(blk(0) < n_full)
        def _():
            start_read(blk(0), 0)

        @pl.when(blk(1) < n_full)
        def _():
            start_read(blk(1), 1)

        def visit(k, sl):
            i_blk = blk(k)

            @pl.when(i_blk < n_full)
            def _():
                wait_read(i_blk, sl)

                @pl.when(k >= 2)
                def _():
                    wait_write(blk(k - 2), sl)

                transpose(sl)
                start_write(i_blk, sl)

                @pl.when(blk(k + 2) < n_full)
                def _():
                    start_read(blk(k + 2), sl)

        def pairvisit(kk, carry):
            for sl in range(2):
                visit(kk * 2 + sl, sl)
            return carry

        lax.fori_loop(0, (n_vis + 2) // 2, pairvisit, 0, unroll=False)

        def pairdrain(kk, carry):
            for sl in range(2):
                k = kk * 2 + sl

                @pl.when((blk(k) < n_full) & (blk(k + 2) >= n_full))
                def _():
                    wait_write(blk(k), sl)

            return carry

        lax.fori_loop(0, (n_vis + 2) // 2, pairdrain, 0, unroll=False)

        if tail:

            @pl.when(wid == 0)
            def _():
                pltpu.sync_copy(tail_hbm, obuf.at[0, pl.ds(0, tail // 2)])
                pltpu.sync_copy(
                    obuf.at[0, pl.ds(0, tail // 2)],
                    out_hbm.at[pl.ds(n_full * 64, tail // 2)],
                )

    return conv(wt, tail_c)


def _gather(table2, idx):
    N = idx.shape[0]
    n_per_w = N // NW
    n_ch = n_per_w // CH
    NBUF = 2
    LA = 1
    WB = 2

    mesh = plsc.VectorSubcoreMesh(**_MESH)

    @functools.partial(
        pl.kernel,
        mesh=mesh,
        compiler_params=_PARAMS,
        out_type=jax.ShapeDtypeStruct((200, 64, 4096), jnp.float32),
        scratch_types=(
            [
                pltpu.VMEM((n_per_w,), jnp.int32),
                pltpu.VMEM((NBUF, CH), jnp.int32),
                pltpu.VMEM((NBUF, CH, 128), jnp.float32),
                pltpu.VMEM((WB, 64, 128), jnp.float32),
            ]
            + [pltpu.SemaphoreType.DMA] * (NBUF + WB)
        ),
    )
    def gat(tab_hbm, idx_hbm, out_hbm, idx_v, pidx, rows, wbuf, *sems):
        gsem = sems[:NBUF]
        osem = sems[NBUF:]
        wid = lax.axis_index("s") * NC + lax.axis_index("c")
        base = wid * n_per_w
        pltpu.sync_copy(idx_hbm.at[pl.ds(base, n_per_w)], idx_v)

        iota = lax.iota(jnp.int32, 16)
        rowvs = [g * 16 + iota for g in range(8)]

        def start_gather(g, b):
            for q in range(8):
                iv = idx_v[pl.ds(g * CH + q * 16, 16)]
                pidx[b, pl.ds(q * 16, 16)] = iv >> 1
            pltpu.async_copy(tab_hbm.at[pidx.at[b]], rows.at[b], gsem[b])

        def wait_gather(g, b):
            pltpu.make_async_copy(
                tab_hbm.at[pidx.at[b]], rows.at[b], gsem[b]
            ).wait()

        def out_ref(g):
            u = base // CH + g
            s = u // 32
            bb = u % 32
            return out_hbm.at[s, :, pl.ds(bb * CH, CH)]

        def start_write(g, ws):
            pltpu.async_copy(wbuf.at[ws], out_ref(g), osem[ws])

        def wait_write(g, ws):
            pltpu.make_async_copy(wbuf.at[ws], out_ref(g), osem[ws]).wait()

        def transpose(g, b, ws):
            def qbody(q, carry):
                iv = idx_v[pl.ds(g * CH + q * 16, 16)]
                parv = (iv & 1) * 64
                rowv = q * 16 + iota
                for f0 in range(0, 64, 8):
                    diags = [((f0 + u) + iota) & 63 for u in range(8)]
                    vals = [
                        plsc.load_gather(rows.at[b], [rowv, parv + dg])
                        for dg in diags
                    ]
                    for u in range(8):
                        plsc.store_scatter(wbuf.at[ws], [diags[u], rowv], vals[u])
                return carry

            lax.fori_loop(0, 8, qbody, 0, unroll=False)

        def do_visit(g, b, ws, issue, reuse_w, unrolled=False):
            wait_gather(g, b)
            if reuse_w:
                wait_write(g - WB, ws)
            transpose(g, b, ws)
            start_write(g, ws)
            if issue:
                start_gather(g + LA, (b + LA) % NBUF)

        for g in range(LA):
            start_gather(g, g % NBUF)

        for g in range(NBUF):
            do_visit(g, g % NBUF, g % WB, g + LA < n_ch, g >= WB)

        def block(blkk, carry):
            for b in range(NBUF):
                g = blkk * NBUF + b
                do_visit(g, b, b % WB, True, True, unrolled=True)
            return carry

        lax.fori_loop(1, n_ch // NBUF - 1, block, 0, unroll=False)

        for g in range(n_ch - NBUF, n_ch):
            do_visit(g, g % NBUF, g % WB, g + LA < n_ch, True)

        for g in range(n_ch - WB, n_ch):
            wait_write(g, g % WB)

    return gat(table2, idx)


def kernel(batch_sent_input, embed_weight):
    B, S = batch_sent_input.shape
    ids = jnp.transpose(batch_sent_input).reshape(B * S).astype(jnp.int32)
    wt = jnp.transpose(embed_weight)
    V, D = embed_weight.shape
    tail = V % 128
    tail_c = jnp.reshape(embed_weight[V - tail :, :], (tail // 2, 2 * D))
    table2 = _compact_table(wt, tail_c)
    out_p = _gather(table2, ids)
    return jnp.transpose(out_p, (2, 0, 1))


# speedup vs baseline: 3.2677x; 1.0077x over previous
import functools

import jax
import jax.numpy as jnp
from jax import lax
from jax.experimental import pallas as pl
from jax.experimental.pallas import tpu as pltpu
from jax.experimental.pallas import tpu_sc as plsc

NC = 2
NS = 16
NW = NC * NS
CH = 128

_PARAMS = pltpu.CompilerParams(use_tc_tiling_on_sc=True, needs_layout_passes=False)
_MESH = dict(core_axis_name="c", subcore_axis_name="s")


def _compact_table(wt, tail_c):
    D, V = wt.shape
    n_full = V // 128
    tail = V - n_full * 128
    Vp = n_full * 128 + (256 if tail else 0)
    per_w = n_full // NW

    mesh = plsc.VectorSubcoreMesh(**_MESH)

    @functools.partial(
        pl.kernel,
        mesh=mesh,
        compiler_params=_PARAMS,
        out_type=jax.ShapeDtypeStruct((Vp // 2, 128), jnp.float32),
        scratch_types=(
            [pltpu.VMEM((2, D, 128), jnp.float32)]
            + [pltpu.VMEM((2, 64, 128), jnp.float32)]
            + [pltpu.SemaphoreType.DMA] * 4
        ),
    )
    def conv(wt_hbm, tail_hbm, out_hbm, tbuf, obuf, tsem0, tsem1, osem0, osem1):
        wid = lax.axis_index("s") * NC + lax.axis_index("c")
        tsem = (tsem0, tsem1)
        osem = (osem0, osem1)

        def blk(k):
            return k * NW + wid

        n_vis = per_w + 1

        def start_read(i_blk, sl):
            pltpu.async_copy(
                wt_hbm.at[:, pl.ds(i_blk * 128, 128)], tbuf.at[sl], tsem[sl]
            )

        def wait_read(i_blk, sl):
            pltpu.make_async_copy(
                wt_hbm.at[:, pl.ds(i_blk * 128, 128)], tbuf.at[sl], tsem[sl]
            ).wait()

        def start_write(i_blk, sl):
            pltpu.async_copy(
                obuf.at[sl], out_hbm.at[pl.ds(i_blk * 64, 64)], osem[sl]
            )

        def wait_write(i_blk, sl):
            pltpu.make_async_copy(
                obuf.at[sl], out_hbm.at[pl.ds(i_blk * 64, 64)], osem[sl]
            ).wait()

        iota = lax.iota(jnp.int32, 16)
        parv = (iota & 1) * 64
        rowvs = [(g * 16 + iota) >> 1 for g in range(8)]
        colrs = [g * 16 + iota for g in range(8)]

        def transpose(sl):
            def fbody(f, carry):
                diag = (f + iota) & (D - 1)
                colw = parv + diag
                vals = [
                    plsc.load_gather(tbuf.at[sl], [diag, colrs[g]])
                    for g in range(8)
                ]
                for g in range(8):
                    plsc.store_scatter(obuf.at[sl], [rowvs[g], colw], vals[g])
                return carry

            lax.fori_loop(0, D, fbody, 0, unroll=False)

        @pl.when(blk(0) < n_full)
        def _():
            start_read(blk(0), 0)

        @pl.when(blk(1) < n_full)
        def _():
            start_read(blk(1), 1)

        def visit(k, sl):
            i_blk = blk(k)

            @pl.when(i_blk < n_full)
            def _():
                wait_read(i_blk, sl)

                @pl.when(k >= 2)
                def _():
                    wait_write(blk(k - 2), sl)

                transpose(sl)
                start_write(i_blk, sl)

                @pl.when(blk(k + 2) < n_full)
                def _():
                    start_read(blk(k + 2), sl)

        def pairvisit(kk, carry):
            for sl in range(2):
                visit(kk * 2 + sl, sl)
            return carry

        lax.fori_loop(0, (n_vis + 2) // 2, pairvisit, 0, unroll=False)

        def pairdrain(kk, carry):
            for sl in range(2):
                k = kk * 2 + sl

                @pl.when((blk(k) < n_full) & (blk(k + 2) >= n_full))
                def _():
                    wait_write(blk(k), sl)

            return carry

        lax.fori_loop(0, (n_vis + 2) // 2, pairdrain, 0, unroll=False)

        if tail:

            @pl.when(wid == 0)
            def _():
                pltpu.sync_copy(tail_hbm, obuf.at[0, pl.ds(0, tail // 2)])
                pltpu.sync_copy(
                    obuf.at[0, pl.ds(0, tail // 2)],
                    out_hbm.at[pl.ds(n_full * 64, tail // 2)],
                )

    return conv(wt, tail_c)


def _gather(table2, idx):
    N = idx.shape[0]
    n_per_w = N // NW
    n_ch = n_per_w // CH
    NBUF = 2
    LA = 1
    WB = 2

    mesh = plsc.VectorSubcoreMesh(**_MESH)

    @functools.partial(
        pl.kernel,
        mesh=mesh,
        compiler_params=_PARAMS,
        out_type=jax.ShapeDtypeStruct((200, 64, 4096), jnp.float32),
        scratch_types=(
            [
                pltpu.VMEM((n_per_w,), jnp.int32),
                pltpu.VMEM((NBUF, CH), jnp.int32),
                pltpu.VMEM((NBUF, CH, 128), jnp.float32),
                pltpu.VMEM((WB, 64, 128), jnp.float32),
            ]
            + [pltpu.SemaphoreType.DMA] * (NBUF + WB)
        ),
    )
    def gat(tab_hbm, idx_hbm, out_hbm, idx_v, pidx, rows, wbuf, *sems):
        gsem = sems[:NBUF]
        osem = sems[NBUF:]
        wid = lax.axis_index("s") * NC + lax.axis_index("c")
        base = wid * n_per_w
        pltpu.sync_copy(idx_hbm.at[pl.ds(base, n_per_w)], idx_v)

        iota = lax.iota(jnp.int32, 16)
        rowvs = [g * 16 + iota for g in range(8)]

        def start_gather(g, b):
            for q in range(8):
                iv = idx_v[pl.ds(g * CH + q * 16, 16)]
                pidx[b, pl.ds(q * 16, 16)] = iv >> 1
            pltpu.async_copy(tab_hbm.at[pidx.at[b]], rows.at[b], gsem[b])

        def wait_gather(g, b):
            pltpu.make_async_copy(
                tab_hbm.at[pidx.at[b]], rows.at[b], gsem[b]
            ).wait()

        def out_ref(g):
            u = base // CH + g
            s = u // 32
            bb = u % 32
            return out_hbm.at[s, :, pl.ds(bb * CH, CH)]

        def start_write(g, ws):
            pltpu.async_copy(wbuf.at[ws], out_ref(g), osem[ws])

        def wait_write(g, ws):
            pltpu.make_async_copy(wbuf.at[ws], out_ref(g), osem[ws]).wait()

        def transpose(g, b, ws):
            def qbody(q, carry):
                iv = idx_v[pl.ds(g * CH + q * 16, 16)]
                parv = (iv & 1) * 64
                rowv = q * 16 + iota
                for f0 in range(0, 64, 16):
                    diags = [((f0 + u) + iota) & 63 for u in range(16)]
                    vals = [
                        plsc.load_gather(rows.at[b], [rowv, parv + dg])
                        for dg in diags
                    ]
                    for u in range(16):
                        plsc.store_scatter(wbuf.at[ws], [diags[u], rowv], vals[u])
                return carry

            lax.fori_loop(0, 8, qbody, 0, unroll=False)

        def do_visit(g, b, ws, issue, reuse_w, unrolled=False):
            wait_gather(g, b)
            if reuse_w:
                wait_write(g - WB, ws)
            transpose(g, b, ws)
            start_write(g, ws)
            if issue:
                start_gather(g + LA, (b + LA) % NBUF)

        for g in range(LA):
            start_gather(g, g % NBUF)

        for g in range(NBUF):
            do_visit(g, g % NBUF, g % WB, g + LA < n_ch, g >= WB)

        def block(blkk, carry):
            for b in range(NBUF):
                g = blkk * NBUF + b
                do_visit(g, b, b % WB, True, True, unrolled=True)
            return carry

        lax.fori_loop(1, n_ch // NBUF - 1, block, 0, unroll=False)

        for g in range(n_ch - NBUF, n_ch):
            do_visit(g, g % NBUF, g % WB, g + LA < n_ch, True)

        for g in range(n_ch - WB, n_ch):
            wait_write(g, g % WB)

    return gat(table2, idx)


def kernel(batch_sent_input, embed_weight):
    B, S = batch_sent_input.shape
    ids = jnp.transpose(batch_sent_input).reshape(B * S).astype(jnp.int32)
    wt = jnp.transpose(embed_weight)
    V, D = embed_weight.shape
    tail = V % 128
    tail_c = jnp.reshape(embed_weight[V - tail :, :], (tail // 2, 2 * D))
    table2 = _compact_table(wt, tail_c)
    out_p = _gather(table2, ids)
    return jnp.transpose(out_p, (2, 0, 1))


# speedup vs baseline: 4.7051x; 1.4399x over previous
import functools

import jax
import jax.numpy as jnp
from jax import lax
from jax.experimental import pallas as pl
from jax.experimental.pallas import tpu as pltpu
from jax.experimental.pallas import tpu_sc as plsc

NC = 2
NS = 16
NW = NC * NS
CH = 128

_PARAMS = pltpu.CompilerParams(use_tc_tiling_on_sc=True, needs_layout_passes=False)
_MESH = dict(core_axis_name="c", subcore_axis_name="s")


def _compact_table(wt, tail_c):
    D, V = wt.shape
    n_full = V // 128
    tail = V - n_full * 128
    Vp = n_full * 128 + (256 if tail else 0)
    per_w = n_full // NW

    mesh = plsc.VectorSubcoreMesh(**_MESH)

    @functools.partial(
        pl.kernel,
        mesh=mesh,
        compiler_params=_PARAMS,
        out_type=jax.ShapeDtypeStruct((Vp // 2, 128), jnp.float32),
        scratch_types=(
            [pltpu.VMEM((2, D, 128), jnp.float32)]
            + [pltpu.VMEM((2, 64, 128), jnp.float32)]
            + [pltpu.SemaphoreType.DMA] * 4
        ),
    )
    def conv(wt_hbm, tail_hbm, out_hbm, tbuf, obuf, tsem0, tsem1, osem0, osem1):
        wid = lax.axis_index("s") * NC + lax.axis_index("c")
        tsem = (tsem0, tsem1)
        osem = (osem0, osem1)

        def blk(k):
            return k * NW + wid

        n_vis = per_w + 1

        def start_read(i_blk, sl):
            pltpu.async_copy(
                wt_hbm.at[:, pl.ds(i_blk * 128, 128)], tbuf.at[sl], tsem[sl]
            )

        def wait_read(i_blk, sl):
            pltpu.make_async_copy(
                wt_hbm.at[:, pl.ds(i_blk * 128, 128)], tbuf.at[sl], tsem[sl]
            ).wait()

        def start_write(i_blk, sl):
            pltpu.async_copy(
                obuf.at[sl], out_hbm.at[pl.ds(i_blk * 64, 64)], osem[sl]
            )

        def wait_write(i_blk, sl):
            pltpu.make_async_copy(
                obuf.at[sl], out_hbm.at[pl.ds(i_blk * 64, 64)], osem[sl]
            ).wait()

        iota = lax.iota(jnp.int32, 16)
        parv = (iota & 1) * 64
        rowvs = [(g * 16 + iota) >> 1 for g in range(8)]
        colrs = [g * 16 + iota for g in range(8)]

        def transpose(sl):
            def fbody(f, carry):
                diag = (f + iota) & (D - 1)
                colw = parv + diag
                vals = [
                    plsc.load_gather(tbuf.at[sl], [diag, colrs[g]])
                    for g in range(8)
                ]
                for g in range(8):
                    plsc.store_scatter(obuf.at[sl], [rowvs[g], colw], vals[g])
                return carry

            lax.fori_loop(0, D, fbody, 0, unroll=False)

        @pl.when(blk(0) < n_full)
        def _():
            start_read(blk(0), 0)

        @pl.when(blk(1) < n_full)
        def _():
            start_read(blk(1), 1)

        def visit(k, sl):
            i_blk = blk(k)

            @pl.when(i_blk < n_full)
            def _():
                wait_read(i_blk, sl)

                @pl.when(k >= 2)
                def _():
                    wait_write(blk(k - 2), sl)

                transpose(sl)
                start_write(i_blk, sl)

                @pl.when(blk(k + 2) < n_full)
                def _():
                    start_read(blk(k + 2), sl)

        def pairvisit(kk, carry):
            for sl in range(2):
                visit(kk * 2 + sl, sl)
            return carry

        lax.fori_loop(0, (n_vis + 2) // 2, pairvisit, 0, unroll=False)

        def pairdrain(kk, carry):
            for sl in range(2):
                k = kk * 2 + sl

                @pl.when((blk(k) < n_full) & (blk(k + 2) >= n_full))
                def _():
                    wait_write(blk(k), sl)

            return carry

        lax.fori_loop(0, (n_vis + 2) // 2, pairdrain, 0, unroll=False)

        if tail:

            @pl.when(wid == 0)
            def _():
                pltpu.sync_copy(tail_hbm, obuf.at[0, pl.ds(0, tail // 2)])
                pltpu.sync_copy(
                    obuf.at[0, pl.ds(0, tail // 2)],
                    out_hbm.at[pl.ds(n_full * 64, tail // 2)],
                )

    return conv(wt, tail_c)


def _gather(table2, idx):
    N = idx.shape[0]
    n_per_w = N // NW
    n_ch = n_per_w // CH
    NBUF = 4
    LA = 2
    WB = 2

    mesh = plsc.VectorSubcoreMesh(**_MESH)

    @functools.partial(
        pl.kernel,
        mesh=mesh,
        compiler_params=_PARAMS,
        out_type=jax.ShapeDtypeStruct((200, 64, 4096), jnp.float32),
        scratch_types=(
            [
                pltpu.VMEM((n_per_w,), jnp.int32),
                pltpu.VMEM((NBUF, CH), jnp.int32),
                pltpu.VMEM((NBUF, CH, 128), jnp.float32),
                pltpu.VMEM((WB, 64, 128), jnp.float32),
            ]
            + [pltpu.SemaphoreType.DMA] * (NBUF + WB)
        ),
    )
    def gat(tab_hbm, idx_hbm, out_hbm, idx_v, pidx, rows, wbuf, *sems):
        gsem = sems[:NBUF]
        osem = sems[NBUF:]
        wid = lax.axis_index("s") * NC + lax.axis_index("c")
        base = wid * n_per_w
        pltpu.sync_copy(idx_hbm.at[pl.ds(base, n_per_w)], idx_v)

        iota = lax.iota(jnp.int32, 16)
        rowvs = [g * 16 + iota for g in range(8)]

        def start_gather(g, b):
            for q in range(8):
                iv = idx_v[pl.ds(g * CH + q * 16, 16)]
                pidx[b, pl.ds(q * 16, 16)] = iv >> 1
            pltpu.async_copy(tab_hbm.at[pidx.at[b]], rows.at[b], gsem[b])

        def wait_gather(g, b):
            pltpu.make_async_copy(
                tab_hbm.at[pidx.at[b]], rows.at[b], gsem[b]
            ).wait()

        def out_ref(g):
            u = base // CH + g
            s = u // 32
            bb = u % 32
            return out_hbm.at[s, :, pl.ds(bb * CH, CH)]

        def start_write(g, ws):
            pltpu.async_copy(wbuf.at[ws], out_ref(g), osem[ws])

        def wait_write(g, ws):
            pltpu.make_async_copy(wbuf.at[ws], out_ref(g), osem[ws]).wait()

        def transpose(g, b, ws):
            def qbody(q, carry):
                iv = idx_v[pl.ds(g * CH + q * 16, 16)]
                parv = (iv & 1) * 64
                rowv = q * 16 + iota
                for f0 in range(0, 64, 16):
                    diags = [((f0 + u) + iota) & 63 for u in range(16)]
                    vals = [
                        plsc.load_gather(rows.at[b], [rowv, parv + dg])
                        for dg in diags
                    ]
                    for u in range(16):
                        plsc.store_scatter(wbuf.at[ws], [diags[u], rowv], vals[u])
                return carry

            lax.fori_loop(0, 8, qbody, 0, unroll=False)

        def do_visit(g, b, ws, issue, reuse_w, unrolled=False):
            wait_gather(g, b)
            if reuse_w:
                wait_write(g - WB, ws)
            transpose(g, b, ws)
            start_write(g, ws)
            if issue:
                start_gather(g + LA, (b + LA) % NBUF)

        for g in range(LA):
            start_gather(g, g % NBUF)

        for g in range(NBUF):
            do_visit(g, g % NBUF, g % WB, g + LA < n_ch, g >= WB)

        def block(blkk, carry):
            for b in range(NBUF):
                g = blkk * NBUF + b
                do_visit(g, b, b % WB, True, True, unrolled=True)
            return carry

        lax.fori_loop(1, n_ch // NBUF - 1, block, 0, unroll=False)

        for g in range(n_ch - NBUF, n_ch):
            do_visit(g, g % NBUF, g % WB, g + LA < n_ch, True)

        for g in range(n_ch - WB, n_ch):
            wait_write(g, g % WB)

    return gat(table2, idx)


def kernel(batch_sent_input, embed_weight):
    B, S = batch_sent_input.shape
    ids = jnp.transpose(batch_sent_input).reshape(B * S).astype(jnp.int32)
    wt = jnp.transpose(embed_weight)
    V, D = embed_weight.shape
    tail = V % 128
    tail_c = jnp.reshape(embed_weight[V - tail :, :], (tail // 2, 2 * D))
    table2 = _compact_table(wt, tail_c)
    out_p = _gather(table2, ids)
    return jnp.transpose(out_p, (2, 0, 1))
